# Initial kernel scaffold; baseline (speedup 1.0000x reference)
import functools

import jax
import jax.numpy as jnp
from jax import lax
from jax.experimental import pallas as pl
from jax.experimental.pallas import tpu as pltpu
from jax.experimental.pallas import tpu_sc as plsc

NU = 50000
NR = 10000
D = 128
ER = 250000
EN = 100000

NUP = 50176
NRP = 10240
ERP = 253952
ENP = 102400
NB_R = ERP // (32 * 128)
NB_N = ENP // (32 * 128)

_MESH = dict(core_axis_name="c", subcore_axis_name="s")


def _fill(ref, rows, width, value):
    vec = jnp.full((16,), value, jnp.float32)

    @pl.loop(0, rows)
    def _(i):
        for j in range(width // 16):
            ref[i, 16 * j:16 * (j + 1)] = vec


def _sc_degrees(su, sd, ns, nd):
    out_type = (
        jax.ShapeDtypeStruct((2, NUP, 16), jnp.float32),
        jax.ShapeDtypeStruct((2, NRP, 16), jnp.float32),
        jax.ShapeDtypeStruct((2, NRP, 16), jnp.float32),
        jax.ShapeDtypeStruct((2, NRP, 16), jnp.float32),
    )
    scratch = [
        pltpu.VMEM_SHARED((NUP, 16), jnp.float32),
        pltpu.VMEM_SHARED((NRP, 16), jnp.float32),
        pltpu.VMEM_SHARED((NRP, 16), jnp.float32),
        pltpu.VMEM_SHARED((NRP, 16), jnp.float32),
        pltpu.VMEM((512, 16), jnp.float32),
        pltpu.VMEM((128, 16), jnp.float32),
        pltpu.VMEM((2, 128), jnp.int32),
    ]

    @functools.partial(
        pl.kernel, out_type=out_type,
        mesh=plsc.VectorSubcoreMesh(**_MESH), scratch_types=scratch)
    def k(su_r, sd_r, ns_r, nd_r, du_p, dr_p, dns_p, dnd_p,
          hu, hr1, hr2, hr3, zb, ones_v, idx):
        c = lax.axis_index("c")
        s = lax.axis_index("s")
        w = c * 16 + s
        _fill(zb, 512, 16, 0.0)
        _fill(ones_v, 128, 16, 1.0)

        @pl.loop(0, 7)
        def _(kk):
            ch = kk * 16 + s

            @pl.when(ch < NUP // 512)
            def _():
                pltpu.sync_copy(zb, hu.at[pl.ds(ch * 512, 512)])

        @pl.loop(0, 2)
        def _(kk):
            ch = kk * 16 + s

            @pl.when(ch < NRP // 512)
            def _():
                pltpu.sync_copy(zb, hr1.at[pl.ds(ch * 512, 512)])
                pltpu.sync_copy(zb, hr2.at[pl.ds(ch * 512, 512)])
                pltpu.sync_copy(zb, hr3.at[pl.ds(ch * 512, 512)])

        plsc.subcore_barrier()

        for arr, hist, nbw in ((su_r, hu, NB_R), (sd_r, hr1, NB_R),
                               (ns_r, hr2, NB_N), (nd_r, hr3, NB_N)):
            @pl.loop(0, nbw)
            def _(b, arr=arr, hist=hist, nbw=nbw):
                base = (w * nbw + b) * 128
                pltpu.sync_copy(arr.at[pl.ds(base, 128)], idx.at[0])
                pltpu.sync_copy(ones_v, hist.at[idx.at[0]], add=True)

        plsc.subcore_barrier()
        ru = NUP // 16
        rr = NRP // 16
        pltpu.sync_copy(hu.at[pl.ds(s * ru, ru)], du_p.at[c, pl.ds(s * ru, ru)])
        pltpu.sync_copy(hr1.at[pl.ds(s * rr, rr)], dr_p.at[c, pl.ds(s * rr, rr)])
        pltpu.sync_copy(hr2.at[pl.ds(s * rr, rr)], dns_p.at[c, pl.ds(s * rr, rr)])
        pltpu.sync_copy(hr3.at[pl.ds(s * rr, rr)], dnd_p.at[c, pl.ds(s * rr, rr)])

    return k(su, sd, ns, nd)


def _sc_agg_rest(su, sd, ns, nd, t_rev, t_near):
    out_type = (
        jax.ShapeDtypeStruct((2, NRP, 128), jnp.float32),
        jax.ShapeDtypeStruct((2, NRP, 128), jnp.float32),
    )
    scratch = [
        pltpu.VMEM_SHARED((NRP, 128), jnp.float32),
        pltpu.VMEM((128, 128), jnp.float32),
        pltpu.VMEM((128, 128), jnp.float32),
        pltpu.VMEM((128,), jnp.int32),
        pltpu.VMEM((2, 128), jnp.int32),
        pltpu.SemaphoreType.DMA,
    ]

    @functools.partial(
        pl.kernel, out_type=out_type,
        mesh=plsc.VectorSubcoreMesh(**_MESH), scratch_types=scratch)
    def k(su_r, sd_r, ns_r, nd_r, trev_r, tnear_r, grev, gnear,
          acc, zb, rows, idxg, idxs, sem):
        c = lax.axis_index("c")
        s = lax.axis_index("s")
        w = c * 16 + s
        _fill(zb, 128, 128, 0.0)

        for src_a, dst_a, tbl, out, nbw in (
                (su_r, sd_r, trev_r, grev, NB_R),
                (ns_r, nd_r, tnear_r, gnear, NB_N)):
            @pl.loop(0, NRP // 128 // 16)
            def _(kk):
                ch = kk * 16 + s
                pltpu.sync_copy(zb, acc.at[pl.ds(ch * 128, 128)])

            plsc.subcore_barrier()

            @pl.loop(0, nbw)
            def _(b, src_a=src_a, dst_a=dst_a, tbl=tbl, nbw=nbw):
                base = (w * nbw + b) * 128
                pltpu.sync_copy(src_a.at[pl.ds(base, 128)], idxg)
                pltpu.sync_copy(dst_a.at[pl.ds(base, 128)], idxs.at[0])
                pltpu.async_copy(tbl.at[idxg], rows, sem).wait()
                pltpu.sync_copy(rows, acc.at[idxs.at[0]], add=True)

            plsc.subcore_barrier()
            rr = NRP // 16
            pltpu.sync_copy(acc.at[pl.ds(s * rr, rr)],
                            out.at[c, pl.ds(s * rr, rr)])
            plsc.subcore_barrier()

    return k(su, sd, ns, nd, t_rev, t_near)


def _sc_agg_user(sd, su, t0, t1, t2, t3):
    out_type = jax.ShapeDtypeStruct((2, 4, NUP, 32), jnp.float32)
    scratch = [
        pltpu.VMEM_SHARED((NUP, 32), jnp.float32),
        pltpu.VMEM((512, 32), jnp.float32),
        pltpu.VMEM((128, 32), jnp.float32),
        pltpu.VMEM((128,), jnp.int32),
        pltpu.VMEM((2, 128), jnp.int32),
        pltpu.SemaphoreType.DMA,
    ]

    @functools.partial(
        pl.kernel, out_type=out_type,
        mesh=plsc.VectorSubcoreMesh(**_MESH), scratch_types=scratch)
    def k(sd_r, su_r, t0_r, t1_r, t2_r, t3_r, g_p,
          acc, zb, rows, idxg, idxs, sem):
        c = lax.axis_index("c")
        s = lax.axis_index("s")
        w = c * 16 + s
        _fill(zb, 512, 32, 0.0)

        for p, tbl in enumerate((t0_r, t1_r, t2_r, t3_r)):
            @pl.loop(0, 7)
            def _(kk):
                ch = kk * 16 + s

                @pl.when(ch < NUP // 512)
                def _():
                    pltpu.sync_copy(zb, acc.at[pl.ds(ch * 512, 512)])

            plsc.subcore_barrier()

            @pl.loop(0, NB_R)
            def _(b, tbl=tbl):
                base = (w * NB_R + b) * 128
                pltpu.sync_copy(sd_r.at[pl.ds(base, 128)], idxg)
                pltpu.sync_copy(su_r.at[pl.ds(base, 128)], idxs.at[0])
                pltpu.async_copy(tbl.at[idxg], rows, sem).wait()
                pltpu.sync_copy(rows, acc.at[idxs.at[0]], add=True)

            plsc.subcore_barrier()
            ru = NUP // 16
            pltpu.sync_copy(acc.at[pl.ds(s * ru, ru)],
                            g_p.at[c, p, pl.ds(s * ru, ru)])
            plsc.subcore_barrier()

    return k(sd, su, t0, t1, t2, t3)


def _au_of(d_ref):
    d = d_ref[0] + d_ref[1]
    return lax.rsqrt(jnp.maximum(d[:, 0], 1.0))


def _an_of(d_ref):
    d = d_ref[0] + d_ref[1]
    return lax.rsqrt(d[:, 0] + 1.0)


def _tc_user_in(x, w, du_p):
    def body(x_ref, w_ref, d_ref, o_ref):
        au = _au_of(d_ref)
        h = jnp.dot(x_ref[...], w_ref[...], preferred_element_type=jnp.float32)
        o_ref[...] = h * au[:, None]

    return pl.pallas_call(
        body, grid=(NUP // 512,),
        in_specs=[pl.BlockSpec((512, 128), lambda i: (i, 0)),
                  pl.BlockSpec((128, 128), lambda i: (0, 0)),
                  pl.BlockSpec((2, 512, 16), lambda i: (0, i, 0))],
        out_specs=pl.BlockSpec((512, 128), lambda i: (i, 0)),
        out_shape=jax.ShapeDtypeStruct((NUP, 128), jnp.float32),
    )(x, w, du_p)


def _rest_outs():
    return (
        [pl.BlockSpec((512, 128), lambda i: (i, 0)),
         pl.BlockSpec((512, 128), lambda i: (i, 0))] +
        [pl.BlockSpec((512, 32), lambda i: (i, 0)) for _ in range(4)],
        [jax.ShapeDtypeStruct((NRP, 128), jnp.float32),
         jax.ShapeDtypeStruct((NRP, 128), jnp.float32)] +
        [jax.ShapeDtypeStruct((NRP, 32), jnp.float32) for _ in range(4)],
    )


def _emit_rest(hr, ar, ans, hr_ref, tn_ref, t0, t1, t2, t3):
    hr_ref[...] = hr
    tn_ref[...] = hr * ans[:, None]
    ha = hr * ar[:, None]
    t0[...] = ha[:, 0:32]
    t1[...] = ha[:, 32:64]
    t2[...] = ha[:, 64:96]
    t3[...] = ha[:, 96:128]


def _tc_rest_in(x, w, dr_p, dns_p):
    def body(x_ref, w_ref, dr_ref, dns_ref, *outs):
        ar = _au_of(dr_ref)
        ans = _an_of(dns_ref)
        h = jnp.dot(x_ref[...], w_ref[...], preferred_element_type=jnp.float32)
        _emit_rest(h, ar, ans, *outs)

    out_specs, out_shape = _rest_outs()
    return pl.pallas_call(
        body, grid=(NRP // 512,),
        in_specs=[pl.BlockSpec((512, 128), lambda i: (i, 0)),
                  pl.BlockSpec((128, 128), lambda i: (0, 0)),
                  pl.BlockSpec((2, 512, 16), lambda i: (0, i, 0)),
                  pl.BlockSpec((2, 512, 16), lambda i: (0, i, 0))],
        out_specs=out_specs,
        out_shape=out_shape,
    )(x, w, dr_p, dns_p)


def _user_g(g_ref):
    g = g_ref[0] + g_ref[1]
    return jnp.concatenate([g[0], g[1], g[2], g[3]], axis=1)


def _tc_user_mid(g_p, du_p, w_rev):
    def body(g_ref, d_ref, w_ref, o_ref):
        au = _au_of(d_ref)
        h = jnp.dot(_user_g(g_ref), w_ref[...],
                    preferred_element_type=jnp.float32)
        hu = jnp.maximum(h * au[:, None], 0.0)
        o_ref[...] = hu * au[:, None]

    return pl.pallas_call(
        body, grid=(NUP // 512,),
        in_specs=[pl.BlockSpec((2, 4, 512, 32), lambda i: (0, 0, i, 0)),
                  pl.BlockSpec((2, 512, 16), lambda i: (0, i, 0)),
                  pl.BlockSpec((128, 128), lambda i: (0, 0))],
        out_specs=pl.BlockSpec((512, 128), lambda i: (i, 0)),
        out_shape=jax.ShapeDtypeStruct((NUP, 128), jnp.float32),
    )(g_p, du_p, w_rev)


def _tc_user_out(g_p, du_p, w_rev, w_out):
    def body(g_ref, d_ref, w_ref, wo_ref, o_ref):
        au = _au_of(d_ref)
        h = jnp.dot(_user_g(g_ref), w_ref[...],
                    preferred_element_type=jnp.float32)
        hu = jnp.maximum(h * au[:, None], 0.0)
        o_ref[...] = jnp.dot(hu, wo_ref[...],
                             preferred_element_type=jnp.float32)

    return pl.pallas_call(
        body, grid=(NUP // 512,),
        in_specs=[pl.BlockSpec((2, 4, 512, 32), lambda i: (0, 0, i, 0)),
                  pl.BlockSpec((2, 512, 16), lambda i: (0, i, 0)),
                  pl.BlockSpec((128, 128), lambda i: (0, 0)),
                  pl.BlockSpec((128, 128), lambda i: (0, 0))],
        out_specs=pl.BlockSpec((512, 128), lambda i: (i, 0)),
        out_shape=jax.ShapeDtypeStruct((NUP, 128), jnp.float32),
    )(g_p, du_p, w_rev, w_out)


def _rest_core(gr_ref, gn_ref, hp_ref, dr_ref, dns_ref, dnd_ref,
               wr_ref, wn_ref):
    ar = _au_of(dr_ref)
    ans = _an_of(dns_ref)
    andd = _an_of(dnd_ref)
    grev = gr_ref[0] + gr_ref[1]
    gnear = gn_ref[0] + gn_ref[1]
    hp = hp_ref[...]
    m1 = jnp.dot(grev * ar[:, None], wr_ref[...],
                 preferred_element_type=jnp.float32)
    near_in = gnear * andd[:, None] + hp * (ans * andd)[:, None]
    m2 = jnp.dot(near_in, wn_ref[...], preferred_element_type=jnp.float32)
    return jnp.maximum(m1 + m2, 0.0), ar, ans


def _rest_in_specs():
    return [pl.BlockSpec((2, 512, 128), lambda i: (0, i, 0)),
            pl.BlockSpec((2, 512, 128), lambda i: (0, i, 0)),
            pl.BlockSpec((512, 128), lambda i: (i, 0)),
            pl.BlockSpec((2, 512, 16), lambda i: (0, i, 0)),
            pl.BlockSpec((2, 512, 16), lambda i: (0, i, 0)),
            pl.BlockSpec((2, 512, 16), lambda i: (0, i, 0)),
            pl.BlockSpec((128, 128), lambda i: (0, 0)),
            pl.BlockSpec((128, 128), lambda i: (0, 0))]


def _tc_rest_mid(grev_p, gnear_p, hr_prev, dr_p, dns_p, dnd_p, w_rev, w_near):
    def body(gr_ref, gn_ref, hp_ref, dr_ref, dns_ref, dnd_ref,
             wr_ref, wn_ref, *outs):
        hr, ar, ans = _rest_core(gr_ref, gn_ref, hp_ref, dr_ref, dns_ref,
                                 dnd_ref, wr_ref, wn_ref)
        _emit_rest(hr, ar, ans, *outs)

    out_specs, out_shape = _rest_outs()
    return pl.pallas_call(
        body, grid=(NRP // 512,),
        in_specs=_rest_in_specs(),
        out_specs=out_specs,
        out_shape=out_shape,
    )(grev_p, gnear_p, hr_prev, dr_p, dns_p, dnd_p, w_rev, w_near)


def _tc_rest_out(grev_p, gnear_p, hr_prev, dr_p, dns_p, dnd_p,
                 w_rev, w_near, w_out):
    def body(gr_ref, gn_ref, hp_ref, dr_ref, dns_ref, dnd_ref,
             wr_ref, wn_ref, wo_ref, o_ref):
        hr, _, _ = _rest_core(gr_ref, gn_ref, hp_ref, dr_ref, dns_ref,
                              dnd_ref, wr_ref, wn_ref)
        o_ref[...] = jnp.dot(hr, wo_ref[...],
                             preferred_element_type=jnp.float32)

    return pl.pallas_call(
        body, grid=(NRP // 512,),
        in_specs=_rest_in_specs() + [pl.BlockSpec((128, 128), lambda i: (0, 0))],
        out_specs=pl.BlockSpec((512, 128), lambda i: (i, 0)),
        out_shape=jax.ShapeDtypeStruct((NRP, 128), jnp.float32),
    )(grev_p, gnear_p, hr_prev, dr_p, dns_p, dnd_p, w_rev, w_near, w_out)


def kernel(x_user, x_restaurant, W_in_user, W_in_rest, W1_reviews, W1_rev,
           W1_near, W2_reviews, W2_rev, W2_near, W_out_user, W_out_rest,
           edge_index_reviews, edge_index_rev_reviews, edge_index_near):
    i32 = jnp.int32
    su = edge_index_reviews[0].astype(i32)
    sd = edge_index_reviews[1].astype(i32)
    ns = edge_index_near[0].astype(i32)
    nd = edge_index_near[1].astype(i32)
    su = jnp.concatenate([su, jnp.full((ERP - ER,), NU, i32)])
    sd = jnp.concatenate([sd, jnp.full((ERP - ER,), NR, i32)])
    ns = jnp.concatenate([ns, jnp.full((ENP - EN,), NR, i32)])
    nd = jnp.concatenate([nd, jnp.full((ENP - EN,), NR, i32)])

    xu = jnp.pad(x_user, ((0, NUP - NU), (0, 0)))
    xr = jnp.pad(x_restaurant, ((0, NRP - NR), (0, 0)))

    du_p, dr_p, dns_p, dnd_p = _sc_degrees(su, sd, ns, nd)

    t_rev1 = _tc_user_in(xu, W_in_user, du_p)
    hr0, t_near1, c10, c11, c12, c13 = _tc_rest_in(xr, W_in_rest, dr_p, dns_p)

    grev1, gnear1 = _sc_agg_rest(su, sd, ns, nd, t_rev1, t_near1)
    gu1 = _sc_agg_user(sd, su, c10, c11, c12, c13)

    t_rev2 = _tc_user_mid(gu1, du_p, W1_rev)
    hr1, t_near2, c20, c21, c22, c23 = _tc_rest_mid(
        grev1, gnear1, hr0, dr_p, dns_p, dnd_p, W1_reviews, W1_near)

    grev2, gnear2 = _sc_agg_rest(su, sd, ns, nd, t_rev2, t_near2)
    gu2 = _sc_agg_user(sd, su, c20, c21, c22, c23)

    out_u = _tc_user_out(gu2, du_p, W2_rev, W_out_user)
    out_r = _tc_rest_out(grev2, gnear2, hr1, dr_p, dns_p, dnd_p,
                         W2_reviews, W2_near, W_out_rest)

    return (out_u[:NU], out_r[:NR])


# speedup vs baseline: 4.9026x; 4.9026x over previous
import functools

import jax
import jax.numpy as jnp
from jax import lax
from jax.experimental import pallas as pl
from jax.experimental.pallas import tpu as pltpu
from jax.experimental.pallas import tpu_sc as plsc

NU = 50000
NR = 10000
D = 128
ER = 250000
EN = 100000

NUP = 50176
NRP = 10240
ERP = 253952
ENP = 102400
NB_R = ERP // (32 * 128)
NB_N = ENP // (32 * 128)

_MESH = dict(core_axis_name="c", subcore_axis_name="s")


def _fill(ref, rows, width, value):
    vec = jnp.full((16,), value, jnp.float32)

    @pl.loop(0, rows)
    def _(i):
        for j in range(width // 16):
            ref[i, 16 * j:16 * (j + 1)] = vec


def _sc_degrees(su, sd, ns, nd):
    out_type = (
        jax.ShapeDtypeStruct((2, NUP, 16), jnp.float32),
        jax.ShapeDtypeStruct((2, NRP, 16), jnp.float32),
        jax.ShapeDtypeStruct((2, NRP, 16), jnp.float32),
        jax.ShapeDtypeStruct((2, NRP, 16), jnp.float32),
    )
    scratch = [
        pltpu.VMEM_SHARED((NUP, 16), jnp.float32),
        pltpu.VMEM_SHARED((NRP, 16), jnp.float32),
        pltpu.VMEM_SHARED((NRP, 16), jnp.float32),
        pltpu.VMEM_SHARED((NRP, 16), jnp.float32),
        pltpu.VMEM((512, 16), jnp.float32),
        pltpu.VMEM((128, 16), jnp.float32),
        pltpu.VMEM((2, 128), jnp.int32),
    ]

    @functools.partial(
        pl.kernel, out_type=out_type,
        mesh=plsc.VectorSubcoreMesh(**_MESH), scratch_types=scratch,
        compiler_params=pltpu.CompilerParams(use_tc_tiling_on_sc=False))
    def k(su_r, sd_r, ns_r, nd_r, du_p, dr_p, dns_p, dnd_p,
          hu, hr1, hr2, hr3, zb, ones_v, idx):
        c = lax.axis_index("c")
        s = lax.axis_index("s")
        w = c * 16 + s
        _fill(zb, 512, 16, 0.0)
        _fill(ones_v, 128, 16, 1.0)

        @pl.loop(0, 7)
        def _(kk):
            ch = kk * 16 + s

            @pl.when(ch < NUP // 512)
            def _():
                pltpu.sync_copy(zb, hu.at[pl.ds(ch * 512, 512)])

        @pl.loop(0, 2)
        def _(kk):
            ch = kk * 16 + s

            @pl.when(ch < NRP // 512)
            def _():
                pltpu.sync_copy(zb, hr1.at[pl.ds(ch * 512, 512)])
                pltpu.sync_copy(zb, hr2.at[pl.ds(ch * 512, 512)])
                pltpu.sync_copy(zb, hr3.at[pl.ds(ch * 512, 512)])

        plsc.subcore_barrier()

        for arr, hist, nbw in ((su_r, hu, NB_R), (sd_r, hr1, NB_R),
                               (ns_r, hr2, NB_N), (nd_r, hr3, NB_N)):
            @pl.loop(0, nbw)
            def _(b, arr=arr, hist=hist, nbw=nbw):
                base = (w * nbw + b) * 128
                pltpu.sync_copy(arr.at[pl.ds(base, 128)], idx.at[0])
                pltpu.sync_copy(ones_v, hist.at[idx.at[0]], add=True)

        plsc.subcore_barrier()
        ru = NUP // 16
        rr = NRP // 16
        pltpu.sync_copy(hu.at[pl.ds(s * ru, ru)], du_p.at[c, pl.ds(s * ru, ru)])
        pltpu.sync_copy(hr1.at[pl.ds(s * rr, rr)], dr_p.at[c, pl.ds(s * rr, rr)])
        pltpu.sync_copy(hr2.at[pl.ds(s * rr, rr)], dns_p.at[c, pl.ds(s * rr, rr)])
        pltpu.sync_copy(hr3.at[pl.ds(s * rr, rr)], dnd_p.at[c, pl.ds(s * rr, rr)])

    return k(su, sd, ns, nd)


def _sc_agg_rest(su, sd, ns, nd, t_rev, t_near):
    out_type = (
        jax.ShapeDtypeStruct((2, NRP, 128), jnp.float32),
        jax.ShapeDtypeStruct((2, NRP, 128), jnp.float32),
    )
    scratch = [
        pltpu.VMEM_SHARED((NRP, 128), jnp.float32),
        pltpu.VMEM((128, 128), jnp.float32),
        pltpu.VMEM((128, 128), jnp.float32),
        pltpu.VMEM((128,), jnp.int32),
        pltpu.VMEM((2, 128), jnp.int32),
        pltpu.SemaphoreType.DMA,
    ]

    @functools.partial(
        pl.kernel, out_type=out_type,
        mesh=plsc.VectorSubcoreMesh(**_MESH), scratch_types=scratch,
        compiler_params=pltpu.CompilerParams(use_tc_tiling_on_sc=False))
    def k(su_r, sd_r, ns_r, nd_r, trev_r, tnear_r, grev, gnear,
          acc, zb, rows, idxg, idxs, sem):
        c = lax.axis_index("c")
        s = lax.axis_index("s")
        w = c * 16 + s
        _fill(zb, 128, 128, 0.0)

        for src_a, dst_a, tbl, out, nbw in (
                (su_r, sd_r, trev_r, grev, NB_R),
                (ns_r, nd_r, tnear_r, gnear, NB_N)):
            @pl.loop(0, NRP // 128 // 16)
            def _(kk):
                ch = kk * 16 + s
                pltpu.sync_copy(zb, acc.at[pl.ds(ch * 128, 128)])

            plsc.subcore_barrier()

            @pl.loop(0, nbw)
            def _(b, src_a=src_a, dst_a=dst_a, tbl=tbl, nbw=nbw):
                base = (w * nbw + b) * 128
                pltpu.sync_copy(src_a.at[pl.ds(base, 128)], idxg)
                pltpu.sync_copy(dst_a.at[pl.ds(base, 128)], idxs.at[0])
                pltpu.async_copy(tbl.at[idxg], rows, sem).wait()
                pltpu.sync_copy(rows, acc.at[idxs.at[0]], add=True)

            plsc.subcore_barrier()
            rr = NRP // 16
            pltpu.sync_copy(acc.at[pl.ds(s * rr, rr)],
                            out.at[c, pl.ds(s * rr, rr)])
            plsc.subcore_barrier()

    return k(su, sd, ns, nd, t_rev, t_near)


def _sc_agg_user(sd, su, t0, t1, t2, t3):
    out_type = jax.ShapeDtypeStruct((2, 4, NUP, 32), jnp.float32)
    scratch = [
        pltpu.VMEM_SHARED((NUP, 32), jnp.float32),
        pltpu.VMEM((512, 32), jnp.float32),
        pltpu.VMEM((128, 32), jnp.float32),
        pltpu.VMEM((128,), jnp.int32),
        pltpu.VMEM((2, 128), jnp.int32),
        pltpu.SemaphoreType.DMA,
    ]

    @functools.partial(
        pl.kernel, out_type=out_type,
        mesh=plsc.VectorSubcoreMesh(**_MESH), scratch_types=scratch,
        compiler_params=pltpu.CompilerParams(use_tc_tiling_on_sc=False))
    def k(sd_r, su_r, t0_r, t1_r, t2_r, t3_r, g_p,
          acc, zb, rows, idxg, idxs, sem):
        c = lax.axis_index("c")
        s = lax.axis_index("s")
        w = c * 16 + s
        _fill(zb, 512, 32, 0.0)

        for p, tbl in enumerate((t0_r, t1_r, t2_r, t3_r)):
            @pl.loop(0, 7)
            def _(kk):
                ch = kk * 16 + s

                @pl.when(ch < NUP // 512)
                def _():
                    pltpu.sync_copy(zb, acc.at[pl.ds(ch * 512, 512)])

            plsc.subcore_barrier()

            @pl.loop(0, NB_R)
            def _(b, tbl=tbl):
                base = (w * NB_R + b) * 128
                pltpu.sync_copy(sd_r.at[pl.ds(base, 128)], idxg)
                pltpu.sync_copy(su_r.at[pl.ds(base, 128)], idxs.at[0])
                pltpu.async_copy(tbl.at[idxg], rows, sem).wait()
                pltpu.sync_copy(rows, acc.at[idxs.at[0]], add=True)

            plsc.subcore_barrier()
            ru = NUP // 16
            pltpu.sync_copy(acc.at[pl.ds(s * ru, ru)],
                            g_p.at[c, p, pl.ds(s * ru, ru)])
            plsc.subcore_barrier()

    return k(sd, su, t0, t1, t2, t3)


def _au_of(d_ref):
    d = d_ref[0] + d_ref[1]
    return lax.rsqrt(jnp.maximum(d[:, 0], 1.0))


def _an_of(d_ref):
    d = d_ref[0] + d_ref[1]
    return lax.rsqrt(d[:, 0] + 1.0)


def _tc_user_in(x, w, du_p):
    def body(x_ref, w_ref, d_ref, o_ref):
        au = _au_of(d_ref)
        h = jnp.dot(x_ref[...], w_ref[...], preferred_element_type=jnp.float32)
        o_ref[...] = h * au[:, None]

    return pl.pallas_call(
        body, grid=(NUP // 512,),
        in_specs=[pl.BlockSpec((512, 128), lambda i: (i, 0)),
                  pl.BlockSpec((128, 128), lambda i: (0, 0)),
                  pl.BlockSpec((2, 512, 16), lambda i: (0, i, 0))],
        out_specs=pl.BlockSpec((512, 128), lambda i: (i, 0)),
        out_shape=jax.ShapeDtypeStruct((NUP, 128), jnp.float32),
    )(x, w, du_p)


def _rest_outs():
    return (
        [pl.BlockSpec((512, 128), lambda i: (i, 0)),
         pl.BlockSpec((512, 128), lambda i: (i, 0))] +
        [pl.BlockSpec((512, 32), lambda i: (i, 0)) for _ in range(4)],
        [jax.ShapeDtypeStruct((NRP, 128), jnp.float32),
         jax.ShapeDtypeStruct((NRP, 128), jnp.float32)] +
        [jax.ShapeDtypeStruct((NRP, 32), jnp.float32) for _ in range(4)],
    )


def _emit_rest(hr, ar, ans, hr_ref, tn_ref, t0, t1, t2, t3):
    hr_ref[...] = hr
    tn_ref[...] = hr * ans[:, None]
    ha = hr * ar[:, None]
    t0[...] = ha[:, 0:32]
    t1[...] = ha[:, 32:64]
    t2[...] = ha[:, 64:96]
    t3[...] = ha[:, 96:128]


def _tc_rest_in(x, w, dr_p, dns_p):
    def body(x_ref, w_ref, dr_ref, dns_ref, *outs):
        ar = _au_of(dr_ref)
        ans = _an_of(dns_ref)
        h = jnp.dot(x_ref[...], w_ref[...], preferred_element_type=jnp.float32)
        _emit_rest(h, ar, ans, *outs)

    out_specs, out_shape = _rest_outs()
    return pl.pallas_call(
        body, grid=(NRP // 512,),
        in_specs=[pl.BlockSpec((512, 128), lambda i: (i, 0)),
                  pl.BlockSpec((128, 128), lambda i: (0, 0)),
                  pl.BlockSpec((2, 512, 16), lambda i: (0, i, 0)),
                  pl.BlockSpec((2, 512, 16), lambda i: (0, i, 0))],
        out_specs=out_specs,
        out_shape=out_shape,
    )(x, w, dr_p, dns_p)


def _user_g(g_ref):
    g = g_ref[0] + g_ref[1]
    return jnp.concatenate([g[0], g[1], g[2], g[3]], axis=1)


def _tc_user_mid(g_p, du_p, w_rev):
    def body(g_ref, d_ref, w_ref, o_ref):
        au = _au_of(d_ref)
        h = jnp.dot(_user_g(g_ref), w_ref[...],
                    preferred_element_type=jnp.float32)
        hu = jnp.maximum(h * au[:, None], 0.0)
        o_ref[...] = hu * au[:, None]

    return pl.pallas_call(
        body, grid=(NUP // 512,),
        in_specs=[pl.BlockSpec((2, 4, 512, 32), lambda i: (0, 0, i, 0)),
                  pl.BlockSpec((2, 512, 16), lambda i: (0, i, 0)),
                  pl.BlockSpec((128, 128), lambda i: (0, 0))],
        out_specs=pl.BlockSpec((512, 128), lambda i: (i, 0)),
        out_shape=jax.ShapeDtypeStruct((NUP, 128), jnp.float32),
    )(g_p, du_p, w_rev)


def _tc_user_out(g_p, du_p, w_rev, w_out):
    def body(g_ref, d_ref, w_ref, wo_ref, o_ref):
        au = _au_of(d_ref)
        h = jnp.dot(_user_g(g_ref), w_ref[...],
                    preferred_element_type=jnp.float32)
        hu = jnp.maximum(h * au[:, None], 0.0)
        o_ref[...] = jnp.dot(hu, wo_ref[...],
                             preferred_element_type=jnp.float32)

    return pl.pallas_call(
        body, grid=(NUP // 512,),
        in_specs=[pl.BlockSpec((2, 4, 512, 32), lambda i: (0, 0, i, 0)),
                  pl.BlockSpec((2, 512, 16), lambda i: (0, i, 0)),
                  pl.BlockSpec((128, 128), lambda i: (0, 0)),
                  pl.BlockSpec((128, 128), lambda i: (0, 0))],
        out_specs=pl.BlockSpec((512, 128), lambda i: (i, 0)),
        out_shape=jax.ShapeDtypeStruct((NUP, 128), jnp.float32),
    )(g_p, du_p, w_rev, w_out)


def _rest_core(gr_ref, gn_ref, hp_ref, dr_ref, dns_ref, dnd_ref,
               wr_ref, wn_ref):
    ar = _au_of(dr_ref)
    ans = _an_of(dns_ref)
    andd = _an_of(dnd_ref)
    grev = gr_ref[0] + gr_ref[1]
    gnear = gn_ref[0] + gn_ref[1]
    hp = hp_ref[...]
    m1 = jnp.dot(grev * ar[:, None], wr_ref[...],
                 preferred_element_type=jnp.float32)
    near_in = gnear * andd[:, None] + hp * (ans * andd)[:, None]
    m2 = jnp.dot(near_in, wn_ref[...], preferred_element_type=jnp.float32)
    return jnp.maximum(m1 + m2, 0.0), ar, ans


def _rest_in_specs():
    return [pl.BlockSpec((2, 512, 128), lambda i: (0, i, 0)),
            pl.BlockSpec((2, 512, 128), lambda i: (0, i, 0)),
            pl.BlockSpec((512, 128), lambda i: (i, 0)),
            pl.BlockSpec((2, 512, 16), lambda i: (0, i, 0)),
            pl.BlockSpec((2, 512, 16), lambda i: (0, i, 0)),
            pl.BlockSpec((2, 512, 16), lambda i: (0, i, 0)),
            pl.BlockSpec((128, 128), lambda i: (0, 0)),
            pl.BlockSpec((128, 128), lambda i: (0, 0))]


def _tc_rest_mid(grev_p, gnear_p, hr_prev, dr_p, dns_p, dnd_p, w_rev, w_near):
    def body(gr_ref, gn_ref, hp_ref, dr_ref, dns_ref, dnd_ref,
             wr_ref, wn_ref, *outs):
        hr, ar, ans = _rest_core(gr_ref, gn_ref, hp_ref, dr_ref, dns_ref,
                                 dnd_ref, wr_ref, wn_ref)
        _emit_rest(hr, ar, ans, *outs)

    out_specs, out_shape = _rest_outs()
    return pl.pallas_call(
        body, grid=(NRP // 512,),
        in_specs=_rest_in_specs(),
        out_specs=out_specs,
        out_shape=out_shape,
    )(grev_p, gnear_p, hr_prev, dr_p, dns_p, dnd_p, w_rev, w_near)


def _tc_rest_out(grev_p, gnear_p, hr_prev, dr_p, dns_p, dnd_p,
                 w_rev, w_near, w_out):
    def body(gr_ref, gn_ref, hp_ref, dr_ref, dns_ref, dnd_ref,
             wr_ref, wn_ref, wo_ref, o_ref):
        hr, _, _ = _rest_core(gr_ref, gn_ref, hp_ref, dr_ref, dns_ref,
                              dnd_ref, wr_ref, wn_ref)
        o_ref[...] = jnp.dot(hr, wo_ref[...],
                             preferred_element_type=jnp.float32)

    return pl.pallas_call(
        body, grid=(NRP // 512,),
        in_specs=_rest_in_specs() + [pl.BlockSpec((128, 128), lambda i: (0, 0))],
        out_specs=pl.BlockSpec((512, 128), lambda i: (i, 0)),
        out_shape=jax.ShapeDtypeStruct((NRP, 128), jnp.float32),
    )(grev_p, gnear_p, hr_prev, dr_p, dns_p, dnd_p, w_rev, w_near, w_out)


def kernel(x_user, x_restaurant, W_in_user, W_in_rest, W1_reviews, W1_rev,
           W1_near, W2_reviews, W2_rev, W2_near, W_out_user, W_out_rest,
           edge_index_reviews, edge_index_rev_reviews, edge_index_near):
    i32 = jnp.int32
    su = edge_index_reviews[0].astype(i32)
    sd = edge_index_reviews[1].astype(i32)
    ns = edge_index_near[0].astype(i32)
    nd = edge_index_near[1].astype(i32)
    su = jnp.concatenate([su, jnp.full((ERP - ER,), NU, i32)])
    sd = jnp.concatenate([sd, jnp.full((ERP - ER,), NR, i32)])
    ns = jnp.concatenate([ns, jnp.full((ENP - EN,), NR, i32)])
    nd = jnp.concatenate([nd, jnp.full((ENP - EN,), NR, i32)])

    xu = jnp.pad(x_user, ((0, NUP - NU), (0, 0)))
    xr = jnp.pad(x_restaurant, ((0, NRP - NR), (0, 0)))

    du_p, dr_p, dns_p, dnd_p = _sc_degrees(su, sd, ns, nd)

    t_rev1 = _tc_user_in(xu, W_in_user, du_p)
    hr0, t_near1, c10, c11, c12, c13 = _tc_rest_in(xr, W_in_rest, dr_p, dns_p)

    grev1, gnear1 = _sc_agg_rest(su, sd, ns, nd, t_rev1, t_near1)
    gu1 = _sc_agg_user(sd, su, c10, c11, c12, c13)

    t_rev2 = _tc_user_mid(gu1, du_p, W1_rev)
    hr1, t_near2, c20, c21, c22, c23 = _tc_rest_mid(
        grev1, gnear1, hr0, dr_p, dns_p, dnd_p, W1_reviews, W1_near)

    grev2, gnear2 = _sc_agg_rest(su, sd, ns, nd, t_rev2, t_near2)
    gu2 = _sc_agg_user(sd, su, c20, c21, c22, c23)

    out_u = _tc_user_out(gu2, du_p, W2_rev, W_out_user)
    out_r = _tc_rest_out(grev2, gnear2, hr1, dr_p, dns_p, dnd_p,
                         W2_reviews, W2_near, W_out_rest)

    return (out_u[:NU], out_r[:NR])


# speedup vs baseline: 6.5209x; 1.3301x over previous
import functools

import jax
import jax.numpy as jnp
from jax import lax
from jax.experimental import pallas as pl
from jax.experimental.pallas import tpu as pltpu
from jax.experimental.pallas import tpu_sc as plsc

NU = 50000
NR = 10000
D = 128
ER = 250000
EN = 100000

NUP = 50176
NRP = 10240
ERP = 253952
ENP = 102400
NB_R = ERP // (32 * 128)
NB_N = ENP // (32 * 128)

_MESH = dict(core_axis_name="c", subcore_axis_name="s")


def _fill(ref, rows, width, value):
    vec = jnp.full((16,), value, jnp.float32)

    @pl.loop(0, rows)
    def _(i):
        for j in range(width // 16):
            ref[i, 16 * j:16 * (j + 1)] = vec


def _zero_shared(zsrc, acc, nchunks, s, chunk):
    @pl.loop(0, (nchunks + 15) // 16)
    def _(kk):
        ch = kk * 16 + s

        @pl.when(ch < nchunks)
        def _():
            pltpu.sync_copy(zsrc, acc.at[pl.ds(ch * chunk, chunk)])


def _sc_degrees(su, sd, ns, nd, z16):
    out_type = (
        jax.ShapeDtypeStruct((2, NUP, 16), jnp.float32),
        jax.ShapeDtypeStruct((2, NRP, 16), jnp.float32),
        jax.ShapeDtypeStruct((2, NRP, 16), jnp.float32),
        jax.ShapeDtypeStruct((2, NRP, 16), jnp.float32),
    )
    scratch = [
        pltpu.VMEM_SHARED((NUP, 16), jnp.float32),
        pltpu.VMEM_SHARED((NRP, 16), jnp.float32),
        pltpu.VMEM_SHARED((NRP, 16), jnp.float32),
        pltpu.VMEM_SHARED((NRP, 16), jnp.float32),
        pltpu.VMEM((128, 16), jnp.float32),
        pltpu.VMEM((NB_R, 128), jnp.int32),
        pltpu.SemaphoreType.DMA,
    ]

    @functools.partial(
        pl.kernel, out_type=out_type,
        mesh=plsc.VectorSubcoreMesh(**_MESH), scratch_types=scratch,
        compiler_params=pltpu.CompilerParams(use_tc_tiling_on_sc=False))
    def k(su_r, sd_r, ns_r, nd_r, z16_r, du_p, dr_p, dns_p, dnd_p,
          hu, hr1, hr2, hr3, ones_v, idx, sem):
        c = lax.axis_index("c")
        s = lax.axis_index("s")
        w = c * 16 + s
        _fill(ones_v, 128, 16, 1.0)
        _zero_shared(z16_r, hu, NUP // 512, s, 512)
        _zero_shared(z16_r, hr1, NRP // 512, s, 512)
        _zero_shared(z16_r, hr2, NRP // 512, s, 512)
        _zero_shared(z16_r, hr3, NRP // 512, s, 512)

        plsc.subcore_barrier()

        for arr, hist, nbw in ((su_r, hu, NB_R), (sd_r, hr1, NB_R),
                               (ns_r, hr2, NB_N), (nd_r, hr3, NB_N)):
            pltpu.sync_copy(arr.at[pl.ds(w * nbw, nbw)], idx.at[pl.ds(0, nbw)])

            @pl.loop(0, nbw, step=8)
            def _(b0, hist=hist, nbw=nbw):
                for j in range(8):
                    @pl.when(b0 + j < nbw)
                    def _(j=j):
                        pltpu.async_copy(ones_v, hist.at[idx.at[b0 + j]],
                                         sem, add=True)
                for j in range(8):
                    @pl.when(b0 + j < nbw)
                    def _(j=j):
                        pltpu.make_async_copy(
                            ones_v, hist.at[idx.at[b0 + j]], sem).wait()

        plsc.subcore_barrier()
        ru = NUP // 16
        rr = NRP // 16
        pltpu.sync_copy(hu.at[pl.ds(s * ru, ru)], du_p.at[c, pl.ds(s * ru, ru)])
        pltpu.sync_copy(hr1.at[pl.ds(s * rr, rr)], dr_p.at[c, pl.ds(s * rr, rr)])
        pltpu.sync_copy(hr2.at[pl.ds(s * rr, rr)], dns_p.at[c, pl.ds(s * rr, rr)])
        pltpu.sync_copy(hr3.at[pl.ds(s * rr, rr)], dnd_p.at[c, pl.ds(s * rr, rr)])

    return k(su, sd, ns, nd, z16)


def _agg_pipeline(tbl, idxg, idxs, rows, acc, sem0, sem1, nbw):
    sems = (sem0, sem1)
    pltpu.async_copy(tbl.at[idxg.at[0]], rows.at[0], sem0)

    @pl.loop(0, (nbw + 1) // 2)
    def _(t):
        for hb in (0, 1):
            b = 2 * t + hb

            @pl.when(b < nbw)
            def _(b=b, hb=hb):
                @pl.when(b + 1 < nbw)
                def _():
                    pltpu.async_copy(tbl.at[idxg.at[b + 1]],
                                     rows.at[1 - hb], sems[1 - hb])
                pltpu.make_async_copy(tbl.at[idxg.at[b]],
                                      rows.at[hb], sems[hb]).wait()
                pltpu.sync_copy(rows.at[hb], acc.at[idxs.at[b]], add=True)


def _sc_agg_rest(su, sd, ns, nd, t_rev, t_near, z128):
    out_type = (
        jax.ShapeDtypeStruct((2, NRP, 128), jnp.float32),
        jax.ShapeDtypeStruct((2, NRP, 128), jnp.float32),
    )
    scratch = [
        pltpu.VMEM_SHARED((NRP, 128), jnp.float32),
        pltpu.VMEM((2, 128, 128), jnp.float32),
        pltpu.VMEM((NB_R, 128), jnp.int32),
        pltpu.VMEM((NB_R, 128), jnp.int32),
        pltpu.SemaphoreType.DMA,
        pltpu.SemaphoreType.DMA,
    ]

    @functools.partial(
        pl.kernel, out_type=out_type,
        mesh=plsc.VectorSubcoreMesh(**_MESH), scratch_types=scratch,
        compiler_params=pltpu.CompilerParams(use_tc_tiling_on_sc=False))
    def k(su_r, sd_r, ns_r, nd_r, trev_r, tnear_r, z128_r, grev, gnear,
          acc, rows, idxg, idxs, sem0, sem1):
        c = lax.axis_index("c")
        s = lax.axis_index("s")
        w = c * 16 + s

        for src_a, dst_a, tbl, out, nbw in (
                (su_r, sd_r, trev_r, grev, NB_R),
                (ns_r, nd_r, tnear_r, gnear, NB_N)):
            _zero_shared(z128_r, acc, NRP // 512, s, 512)
            pltpu.sync_copy(src_a.at[pl.ds(w * nbw, nbw)],
                            idxg.at[pl.ds(0, nbw)])
            pltpu.sync_copy(dst_a.at[pl.ds(w * nbw, nbw)],
                            idxs.at[pl.ds(0, nbw)])
            plsc.subcore_barrier()

            _agg_pipeline(tbl, idxg, idxs, rows, acc, sem0, sem1, nbw)

            plsc.subcore_barrier()
            rr = NRP // 16
            pltpu.sync_copy(acc.at[pl.ds(s * rr, rr)],
                            out.at[c, pl.ds(s * rr, rr)])
            plsc.subcore_barrier()

    return k(su, sd, ns, nd, t_rev, t_near, z128)


def _sc_agg_user(sd, su, t0, t1, t2, t3, z32):
    out_type = jax.ShapeDtypeStruct((2, 4, NUP, 32), jnp.float32)
    scratch = [
        pltpu.VMEM_SHARED((NUP, 32), jnp.float32),
        pltpu.VMEM((2, 128, 32), jnp.float32),
        pltpu.VMEM((NB_R, 128), jnp.int32),
        pltpu.VMEM((NB_R, 128), jnp.int32),
        pltpu.SemaphoreType.DMA,
        pltpu.SemaphoreType.DMA,
    ]

    @functools.partial(
        pl.kernel, out_type=out_type,
        mesh=plsc.VectorSubcoreMesh(**_MESH), scratch_types=scratch,
        compiler_params=pltpu.CompilerParams(use_tc_tiling_on_sc=False))
    def k(sd_r, su_r, t0_r, t1_r, t2_r, t3_r, z32_r, g_p,
          acc, rows, idxg, idxs, sem0, sem1):
        c = lax.axis_index("c")
        s = lax.axis_index("s")
        w = c * 16 + s
        pltpu.sync_copy(sd_r.at[pl.ds(w * NB_R, NB_R)], idxg)
        pltpu.sync_copy(su_r.at[pl.ds(w * NB_R, NB_R)], idxs)

        for p, tbl in enumerate((t0_r, t1_r, t2_r, t3_r)):
            _zero_shared(z32_r, acc, NUP // 512, s, 512)
            plsc.subcore_barrier()

            _agg_pipeline(tbl, idxg, idxs, rows, acc, sem0, sem1, NB_R)

            plsc.subcore_barrier()
            ru = NUP // 16
            pltpu.sync_copy(acc.at[pl.ds(s * ru, ru)],
                            g_p.at[c, p, pl.ds(s * ru, ru)])
            plsc.subcore_barrier()

    return k(sd, su, t0, t1, t2, t3, z32)


def _au_of(d_ref):
    d = d_ref[0] + d_ref[1]
    return lax.rsqrt(jnp.maximum(d[:, 0], 1.0))


def _an_of(d_ref):
    d = d_ref[0] + d_ref[1]
    return lax.rsqrt(d[:, 0] + 1.0)


def _tc_user_in(x, w, du_p):
    def body(x_ref, w_ref, d_ref, o_ref):
        au = _au_of(d_ref)
        h = jnp.dot(x_ref[...], w_ref[...], preferred_element_type=jnp.float32)
        o_ref[...] = h * au[:, None]

    return pl.pallas_call(
        body, grid=(NUP // 512,),
        in_specs=[pl.BlockSpec((512, 128), lambda i: (i, 0)),
                  pl.BlockSpec((128, 128), lambda i: (0, 0)),
                  pl.BlockSpec((2, 512, 16), lambda i: (0, i, 0))],
        out_specs=pl.BlockSpec((512, 128), lambda i: (i, 0)),
        out_shape=jax.ShapeDtypeStruct((NUP, 128), jnp.float32),
    )(x, w, du_p)


def _rest_outs():
    return (
        [pl.BlockSpec((512, 128), lambda i: (i, 0)),
         pl.BlockSpec((512, 128), lambda i: (i, 0))] +
        [pl.BlockSpec((512, 32), lambda i: (i, 0)) for _ in range(4)],
        [jax.ShapeDtypeStruct((NRP, 128), jnp.float32),
         jax.ShapeDtypeStruct((NRP, 128), jnp.float32)] +
        [jax.ShapeDtypeStruct((NRP, 32), jnp.float32) for _ in range(4)],
    )


def _emit_rest(hr, ar, ans, hr_ref, tn_ref, t0, t1, t2, t3):
    hr_ref[...] = hr
    tn_ref[...] = hr * ans[:, None]
    ha = hr * ar[:, None]
    t0[...] = ha[:, 0:32]
    t1[...] = ha[:, 32:64]
    t2[...] = ha[:, 64:96]
    t3[...] = ha[:, 96:128]


def _tc_rest_in(x, w, dr_p, dns_p):
    def body(x_ref, w_ref, dr_ref, dns_ref, *outs):
        ar = _au_of(dr_ref)
        ans = _an_of(dns_ref)
        h = jnp.dot(x_ref[...], w_ref[...], preferred_element_type=jnp.float32)
        _emit_rest(h, ar, ans, *outs)

    out_specs, out_shape = _rest_outs()
    return pl.pallas_call(
        body, grid=(NRP // 512,),
        in_specs=[pl.BlockSpec((512, 128), lambda i: (i, 0)),
                  pl.BlockSpec((128, 128), lambda i: (0, 0)),
                  pl.BlockSpec((2, 512, 16), lambda i: (0, i, 0)),
                  pl.BlockSpec((2, 512, 16), lambda i: (0, i, 0))],
        out_specs=out_specs,
        out_shape=out_shape,
    )(x, w, dr_p, dns_p)


def _user_g(g_ref):
    g = g_ref[0] + g_ref[1]
    return jnp.concatenate([g[0], g[1], g[2], g[3]], axis=1)


def _tc_user_mid(g_p, du_p, w_rev):
    def body(g_ref, d_ref, w_ref, o_ref):
        au = _au_of(d_ref)
        h = jnp.dot(_user_g(g_ref), w_ref[...],
                    preferred_element_type=jnp.float32)
        hu = jnp.maximum(h * au[:, None], 0.0)
        o_ref[...] = hu * au[:, None]

    return pl.pallas_call(
        body, grid=(NUP // 512,),
        in_specs=[pl.BlockSpec((2, 4, 512, 32), lambda i: (0, 0, i, 0)),
                  pl.BlockSpec((2, 512, 16), lambda i: (0, i, 0)),
                  pl.BlockSpec((128, 128), lambda i: (0, 0))],
        out_specs=pl.BlockSpec((512, 128), lambda i: (i, 0)),
        out_shape=jax.ShapeDtypeStruct((NUP, 128), jnp.float32),
    )(g_p, du_p, w_rev)


def _tc_user_out(g_p, du_p, w_rev, w_out):
    def body(g_ref, d_ref, w_ref, wo_ref, o_ref):
        au = _au_of(d_ref)
        h = jnp.dot(_user_g(g_ref), w_ref[...],
                    preferred_element_type=jnp.float32)
        hu = jnp.maximum(h * au[:, None], 0.0)
        o_ref[...] = jnp.dot(hu, wo_ref[...],
                             preferred_element_type=jnp.float32)

    return pl.pallas_call(
        body, grid=(NUP // 512,),
        in_specs=[pl.BlockSpec((2, 4, 512, 32), lambda i: (0, 0, i, 0)),
                  pl.BlockSpec((2, 512, 16), lambda i: (0, i, 0)),
                  pl.BlockSpec((128, 128), lambda i: (0, 0)),
                  pl.BlockSpec((128, 128), lambda i: (0, 0))],
        out_specs=pl.BlockSpec((512, 128), lambda i: (i, 0)),
        out_shape=jax.ShapeDtypeStruct((NUP, 128), jnp.float32),
    )(g_p, du_p, w_rev, w_out)


def _rest_core(gr_ref, gn_ref, hp_ref, dr_ref, dns_ref, dnd_ref,
               wr_ref, wn_ref):
    ar = _au_of(dr_ref)
    ans = _an_of(dns_ref)
    andd = _an_of(dnd_ref)
    grev = gr_ref[0] + gr_ref[1]
    gnear = gn_ref[0] + gn_ref[1]
    hp = hp_ref[...]
    m1 = jnp.dot(grev * ar[:, None], wr_ref[...],
                 preferred_element_type=jnp.float32)
    near_in = gnear * andd[:, None] + hp * (ans * andd)[:, None]
    m2 = jnp.dot(near_in, wn_ref[...], preferred_element_type=jnp.float32)
    return jnp.maximum(m1 + m2, 0.0), ar, ans


def _rest_in_specs():
    return [pl.BlockSpec((2, 512, 128), lambda i: (0, i, 0)),
            pl.BlockSpec((2, 512, 128), lambda i: (0, i, 0)),
            pl.BlockSpec((512, 128), lambda i: (i, 0)),
            pl.BlockSpec((2, 512, 16), lambda i: (0, i, 0)),
            pl.BlockSpec((2, 512, 16), lambda i: (0, i, 0)),
            pl.BlockSpec((2, 512, 16), lambda i: (0, i, 0)),
            pl.BlockSpec((128, 128), lambda i: (0, 0)),
            pl.BlockSpec((128, 128), lambda i: (0, 0))]


def _tc_rest_mid(grev_p, gnear_p, hr_prev, dr_p, dns_p, dnd_p, w_rev, w_near):
    def body(gr_ref, gn_ref, hp_ref, dr_ref, dns_ref, dnd_ref,
             wr_ref, wn_ref, *outs):
        hr, ar, ans = _rest_core(gr_ref, gn_ref, hp_ref, dr_ref, dns_ref,
                                 dnd_ref, wr_ref, wn_ref)
        _emit_rest(hr, ar, ans, *outs)

    out_specs, out_shape = _rest_outs()
    return pl.pallas_call(
        body, grid=(NRP // 512,),
        in_specs=_rest_in_specs(),
        out_specs=out_specs,
        out_shape=out_shape,
    )(grev_p, gnear_p, hr_prev, dr_p, dns_p, dnd_p, w_rev, w_near)


def _tc_rest_out(grev_p, gnear_p, hr_prev, dr_p, dns_p, dnd_p,
                 w_rev, w_near, w_out):
    def body(gr_ref, gn_ref, hp_ref, dr_ref, dns_ref, dnd_ref,
             wr_ref, wn_ref, wo_ref, o_ref):
        hr, _, _ = _rest_core(gr_ref, gn_ref, hp_ref, dr_ref, dns_ref,
                              dnd_ref, wr_ref, wn_ref)
        o_ref[...] = jnp.dot(hr, wo_ref[...],
                             preferred_element_type=jnp.float32)

    return pl.pallas_call(
        body, grid=(NRP // 512,),
        in_specs=_rest_in_specs() + [pl.BlockSpec((128, 128), lambda i: (0, 0))],
        out_specs=pl.BlockSpec((512, 128), lambda i: (i, 0)),
        out_shape=jax.ShapeDtypeStruct((NRP, 128), jnp.float32),
    )(grev_p, gnear_p, hr_prev, dr_p, dns_p, dnd_p, w_rev, w_near, w_out)


def kernel(x_user, x_restaurant, W_in_user, W_in_rest, W1_reviews, W1_rev,
           W1_near, W2_reviews, W2_rev, W2_near, W_out_user, W_out_rest,
           edge_index_reviews, edge_index_rev_reviews, edge_index_near):
    i32 = jnp.int32
    su = edge_index_reviews[0].astype(i32)
    sd = edge_index_reviews[1].astype(i32)
    ns = edge_index_near[0].astype(i32)
    nd = edge_index_near[1].astype(i32)
    su = jnp.concatenate([su, jnp.full((ERP - ER,), NU, i32)]).reshape(-1, 128)
    sd = jnp.concatenate([sd, jnp.full((ERP - ER,), NR, i32)]).reshape(-1, 128)
    ns = jnp.concatenate([ns, jnp.full((ENP - EN,), NR, i32)]).reshape(-1, 128)
    nd = jnp.concatenate([nd, jnp.full((ENP - EN,), NR, i32)]).reshape(-1, 128)

    xu = jnp.pad(x_user, ((0, NUP - NU), (0, 0)))
    xr = jnp.pad(x_restaurant, ((0, NRP - NR), (0, 0)))

    z16 = jnp.zeros((512, 16), jnp.float32)
    z32 = jnp.zeros((512, 32), jnp.float32)
    z128 = jnp.zeros((512, 128), jnp.float32)

    du_p, dr_p, dns_p, dnd_p = _sc_degrees(su, sd, ns, nd, z16)

    t_rev1 = _tc_user_in(xu, W_in_user, du_p)
    hr0, t_near1, c10, c11, c12, c13 = _tc_rest_in(xr, W_in_rest, dr_p, dns_p)

    grev1, gnear1 = _sc_agg_rest(su, sd, ns, nd, t_rev1, t_near1, z128)
    gu1 = _sc_agg_user(sd, su, c10, c11, c12, c13, z32)

    t_rev2 = _tc_user_mid(gu1, du_p, W1_rev)
    hr1, t_near2, c20, c21, c22, c23 = _tc_rest_mid(
        grev1, gnear1, hr0, dr_p, dns_p, dnd_p, W1_reviews, W1_near)

    grev2, gnear2 = _sc_agg_rest(su, sd, ns, nd, t_rev2, t_near2, z128)
    gu2 = _sc_agg_user(sd, su, c20, c21, c22, c23, z32)

    out_u = _tc_user_out(gu2, du_p, W2_rev, W_out_user)
    out_r = _tc_rest_out(grev2, gnear2, hr1, dr_p, dns_p, dnd_p,
                         W2_reviews, W2_near, W_out_rest)

    return (out_u[:NU], out_r[:NR])
